# baseline (speedup 1.0000x reference)
import functools

import jax
import jax.numpy as jnp
from jax import lax
from jax.experimental import pallas as pl
from jax.experimental.pallas import tpu as pltpu
from jax.experimental.pallas import tpu_sc as plsc

_H = 12
_S = 2048
_D = 768
_NW = 32
_CHUNK = 256
_NCHUNK = (_H * _S) // _CHUNK
_CPW = _NCHUNK // _NW
_SUB = 16
_NSUB = _CHUNK // _SUB
_LG = _S // 16


def _sc_colsum_body(probs_hbm, out_hbm, buf0, buf1, acc, sem0, sem1):
    wid = lax.axis_index("s") * 2 + lax.axis_index("c")
    bufs = (buf0, buf1)
    sems = (sem0, sem1)

    for k in range(_CPW):
        chunk = wid * _CPW + k
        base = chunk * _CHUNK

        copies = [None, None]
        copies[0] = pltpu.async_copy(
            probs_hbm.at[pl.ds(base, _SUB)], bufs[0], sems[0])
        for i in range(_NSUB):
            if i + 1 < _NSUB:
                copies[(i + 1) % 2] = pltpu.async_copy(
                    probs_hbm.at[pl.ds(base + (i + 1) * _SUB, _SUB)],
                    bufs[(i + 1) % 2], sems[(i + 1) % 2])
            copies[i % 2].wait()
            buf = bufs[i % 2]

            def lane_group(j, _, buf=buf, i=i, k=k):
                sl = pl.ds(j * 16, 16)
                if i == 0:
                    v = buf[0, sl]
                else:
                    v = acc[k, sl] + buf[0, sl]
                for r in range(1, _SUB):
                    v = v + buf[r, sl]
                acc[k, sl] = v
                return 0

            lax.fori_loop(0, _LG, lane_group, 0)

        pltpu.sync_copy(acc.at[pl.ds(k, 1)], out_hbm.at[pl.ds(chunk, 1)])


_sc_colsum = functools.partial(
    pl.kernel,
    mesh=plsc.VectorSubcoreMesh(core_axis_name="c", subcore_axis_name="s"),
    out_type=jax.ShapeDtypeStruct((_NCHUNK, _S), jnp.float32),
    scratch_types=[
        pltpu.VMEM((_SUB, _S), jnp.float32),
        pltpu.VMEM((_SUB, _S), jnp.float32),
        pltpu.VMEM((_CPW, _S), jnp.float32),
        pltpu.SemaphoreType.DMA,
        pltpu.SemaphoreType.DMA,
    ],
)(_sc_colsum_body)


def _finalize_body(h_ref, cs_ref, w_ref, thr_ref, temp_ref, out_ref):
    imp = jax.lax.dot_general(
        cs_ref[...], w_ref[...],
        dimension_numbers=(((0,), (0,)), ((), ())),
        preferred_element_type=jnp.float32,
    )
    row = jax.lax.broadcasted_iota(jnp.int32, imp.shape, 0)
    imp = jnp.where(row == 0, imp + 100.0, imp)
    mask = jax.nn.sigmoid((imp - thr_ref[0, 0]) / temp_ref[0, 0])
    out_ref[...] = h_ref[...] * mask


def kernel(hidden_states, attention_probs, head_masks, attention_mask, temp, threshold):
    probs = attention_probs.reshape(_H * _S, _S)
    partials = _sc_colsum(probs)

    w = (jnp.repeat(head_masks, _S // _CHUNK) / jnp.sum(head_masks))
    w = w.reshape(_NCHUNK, 1)

    hidden = hidden_states.reshape(_S, _D)
    out = pl.pallas_call(
        _finalize_body,
        in_specs=[
            pl.BlockSpec((_S, _D), lambda: (0, 0)),
            pl.BlockSpec((_NCHUNK, _S), lambda: (0, 0)),
            pl.BlockSpec((_NCHUNK, 1), lambda: (0, 0)),
            pl.BlockSpec((1, 1), lambda: (0, 0)),
            pl.BlockSpec((1, 1), lambda: (0, 0)),
        ],
        out_specs=pl.BlockSpec((_S, _D), lambda: (0, 0)),
        out_shape=jax.ShapeDtypeStruct((_S, _D), jnp.float32),
    )(hidden, partials, w, threshold.reshape(1, 1), temp.reshape(1, 1))

    return (out.reshape(1, _S, _D), attention_mask)


# speedup vs baseline: 1.5172x; 1.5172x over previous
import functools

import jax
import jax.numpy as jnp
from jax import lax
from jax.experimental import pallas as pl
from jax.experimental.pallas import tpu as pltpu
from jax.experimental.pallas import tpu_sc as plsc

_H = 12
_S = 2048
_D = 768
_NW = 32
_CHUNK = 256
_NCHUNK = (_H * _S) // _CHUNK
_CPW = _NCHUNK // _NW
_SUB = 16
_NSUB = _CHUNK // _SUB
_LG = _S // 16


def _sc_colsum_body(probs_hbm, out_hbm, buf0, buf1, acc, sem0, sem1):
    wid = lax.axis_index("s") * 2 + lax.axis_index("c")
    bufs = (buf0, buf1)
    sems = (sem0, sem1)

    for k in range(_CPW):
        chunk = wid * _CPW + k
        base = chunk * _CHUNK

        copies = [None, None]
        copies[0] = pltpu.async_copy(
            probs_hbm.at[pl.ds(base, _SUB)], bufs[0], sems[0])
        for i in range(_NSUB):
            if i + 1 < _NSUB:
                copies[(i + 1) % 2] = pltpu.async_copy(
                    probs_hbm.at[pl.ds(base + (i + 1) * _SUB, _SUB)],
                    bufs[(i + 1) % 2], sems[(i + 1) % 2])
            copies[i % 2].wait()
            buf = bufs[i % 2]

            @plsc.parallel_loop(0, _LG, step=1, unroll=2)
            def lane_group(j, buf=buf, i=i, k=k):
                sl = pl.ds(j * 16, 16)
                rows = [buf[r, sl] for r in range(_SUB)]
                if i > 0:
                    rows.append(acc[k, sl])
                while len(rows) > 1:
                    nxt = [rows[t] + rows[t + 1]
                           for t in range(0, len(rows) - 1, 2)]
                    if len(rows) % 2:
                        nxt.append(rows[-1])
                    rows = nxt
                acc[k, sl] = rows[0]

        pltpu.sync_copy(acc.at[pl.ds(k, 1)], out_hbm.at[pl.ds(chunk, 1)])


_sc_colsum = functools.partial(
    pl.kernel,
    mesh=plsc.VectorSubcoreMesh(core_axis_name="c", subcore_axis_name="s"),
    out_type=jax.ShapeDtypeStruct((_NCHUNK, _S), jnp.float32),
    scratch_types=[
        pltpu.VMEM((_SUB, _S), jnp.float32),
        pltpu.VMEM((_SUB, _S), jnp.float32),
        pltpu.VMEM((_CPW, _S), jnp.float32),
        pltpu.SemaphoreType.DMA,
        pltpu.SemaphoreType.DMA,
    ],
)(_sc_colsum_body)


def _finalize_body(h_ref, cs_ref, w_ref, thr_ref, temp_ref, out_ref):
    imp = jax.lax.dot_general(
        cs_ref[...], w_ref[...],
        dimension_numbers=(((0,), (0,)), ((), ())),
        preferred_element_type=jnp.float32,
    )
    row = jax.lax.broadcasted_iota(jnp.int32, imp.shape, 0)
    imp = jnp.where(row == 0, imp + 100.0, imp)
    mask = jax.nn.sigmoid((imp - thr_ref[0, 0]) / temp_ref[0, 0])
    out_ref[...] = h_ref[...] * mask


def kernel(hidden_states, attention_probs, head_masks, attention_mask, temp, threshold):
    probs = attention_probs.reshape(_H * _S, _S)
    partials = _sc_colsum(probs)

    w = (jnp.repeat(head_masks, _S // _CHUNK) / jnp.sum(head_masks))
    w = w.reshape(_NCHUNK, 1)

    hidden = hidden_states.reshape(_S, _D)
    out = pl.pallas_call(
        _finalize_body,
        in_specs=[
            pl.BlockSpec((_S, _D), lambda: (0, 0)),
            pl.BlockSpec((_NCHUNK, _S), lambda: (0, 0)),
            pl.BlockSpec((_NCHUNK, 1), lambda: (0, 0)),
            pl.BlockSpec((1, 1), lambda: (0, 0)),
            pl.BlockSpec((1, 1), lambda: (0, 0)),
        ],
        out_specs=pl.BlockSpec((_S, _D), lambda: (0, 0)),
        out_shape=jax.ShapeDtypeStruct((_S, _D), jnp.float32),
    )(hidden, partials, w, threshold.reshape(1, 1), temp.reshape(1, 1))

    return (out.reshape(1, _S, _D), attention_mask)


# speedup vs baseline: 1.6917x; 1.1150x over previous
import functools

import jax
import jax.numpy as jnp
from jax import lax
from jax.experimental import pallas as pl
from jax.experimental.pallas import tpu as pltpu
from jax.experimental.pallas import tpu_sc as plsc

_H = 12
_S = 2048
_D = 768
_NW = 32
_CHUNK = 256
_NCHUNK = (_H * _S) // _CHUNK
_NSC = 40
_NTC = _NCHUNK - _NSC
_SC_ROUNDS = (_NSC + _NW - 1) // _NW
_SUB = 16
_NSUB = _CHUNK // _SUB
_LG = _S // 16


def _sc_do_chunk(probs_hbm, out_hbm, bufs, acc, sems, chunk):
    base = chunk * _CHUNK
    copies = [None, None]
    copies[0] = pltpu.async_copy(
        probs_hbm.at[pl.ds(base, _SUB)], bufs[0], sems[0])
    for i in range(_NSUB):
        if i + 1 < _NSUB:
            copies[(i + 1) % 2] = pltpu.async_copy(
                probs_hbm.at[pl.ds(base + (i + 1) * _SUB, _SUB)],
                bufs[(i + 1) % 2], sems[(i + 1) % 2])
        copies[i % 2].wait()
        buf = bufs[i % 2]

        @plsc.parallel_loop(0, _LG, step=1, unroll=4)
        def lane_group(j, buf=buf, i=i):
            sl = pl.ds(j * 16, 16)
            rows = [buf[r, sl] for r in range(_SUB)]
            if i > 0:
                rows.append(acc[0, sl])
            while len(rows) > 1:
                nxt = [rows[t] + rows[t + 1]
                       for t in range(0, len(rows) - 1, 2)]
                if len(rows) % 2:
                    nxt.append(rows[-1])
                rows = nxt
            acc[0, sl] = rows[0]

    pltpu.sync_copy(acc.at[pl.ds(0, 1)], out_hbm.at[pl.ds(chunk, 1)])


def _sc_colsum_body(probs_hbm, out_hbm, buf0, buf1, acc, sem0, sem1):
    wid = lax.axis_index("s") * 2 + lax.axis_index("c")
    for j in range(_SC_ROUNDS):
        chunk = wid + _NW * j
        if (j + 1) * _NW <= _NSC:
            _sc_do_chunk(probs_hbm, out_hbm, (buf0, buf1), acc,
                         (sem0, sem1), chunk)
        else:
            @pl.when(chunk < _NSC)
            def _():
                _sc_do_chunk(probs_hbm, out_hbm, (buf0, buf1), acc,
                             (sem0, sem1), chunk)


_sc_colsum = functools.partial(
    pl.kernel,
    mesh=plsc.VectorSubcoreMesh(core_axis_name="c", subcore_axis_name="s"),
    out_type=jax.ShapeDtypeStruct((_NSC, _S), jnp.float32),
    scratch_types=[
        pltpu.VMEM((_SUB, _S), jnp.float32),
        pltpu.VMEM((_SUB, _S), jnp.float32),
        pltpu.VMEM((1, _S), jnp.float32),
        pltpu.SemaphoreType.DMA,
        pltpu.SemaphoreType.DMA,
    ],
)(_sc_colsum_body)


def _tc_colsum_body(p_ref, out_ref):
    out_ref[...] = jnp.sum(p_ref[...], axis=1, keepdims=True)


def _finalize_body(h_ref, sc_ref, tc_ref, wsc_ref, wtc_ref,
                   thr_ref, temp_ref, out_ref):
    imp = jax.lax.dot_general(
        sc_ref[...], wsc_ref[...],
        dimension_numbers=(((0,), (0,)), ((), ())),
        preferred_element_type=jnp.float32,
    ) + jax.lax.dot_general(
        tc_ref[...], wtc_ref[...],
        dimension_numbers=(((0,), (0,)), ((), ())),
        preferred_element_type=jnp.float32,
    )
    row = jax.lax.broadcasted_iota(jnp.int32, imp.shape, 0)
    imp = jnp.where(row == 0, imp + 100.0, imp)
    mask = jax.nn.sigmoid((imp - thr_ref[0, 0]) / temp_ref[0, 0])
    out_ref[...] = h_ref[...] * mask


def kernel(hidden_states, attention_probs, head_masks, attention_mask, temp, threshold):
    probs = attention_probs.reshape(_H * _S, _S)
    sc_partials = _sc_colsum(probs)

    probs_c = attention_probs.reshape(_NCHUNK, _CHUNK, _S)
    tc_partials = pl.pallas_call(
        _tc_colsum_body,
        grid=(_NTC,),
        in_specs=[pl.BlockSpec((1, _CHUNK, _S), lambda r: (r + _NSC, 0, 0))],
        out_specs=pl.BlockSpec((1, 1, _S), lambda r: (r, 0, 0)),
        out_shape=jax.ShapeDtypeStruct((_NTC, 1, _S), jnp.float32),
        compiler_params=pltpu.CompilerParams(
            dimension_semantics=("parallel",),
        ),
    )(probs_c)

    w = (jnp.repeat(head_masks, _S // _CHUNK) / jnp.sum(head_masks))
    w = w.reshape(_NCHUNK, 1)

    hidden = hidden_states.reshape(_S, _D)
    out = pl.pallas_call(
        _finalize_body,
        in_specs=[
            pl.BlockSpec((_S, _D), lambda: (0, 0)),
            pl.BlockSpec((_NSC, _S), lambda: (0, 0)),
            pl.BlockSpec((_NTC, _S), lambda: (0, 0)),
            pl.BlockSpec((_NSC, 1), lambda: (0, 0)),
            pl.BlockSpec((_NTC, 1), lambda: (0, 0)),
            pl.BlockSpec((1, 1), lambda: (0, 0)),
            pl.BlockSpec((1, 1), lambda: (0, 0)),
        ],
        out_specs=pl.BlockSpec((_S, _D), lambda: (0, 0)),
        out_shape=jax.ShapeDtypeStruct((_S, _D), jnp.float32),
    )(hidden, sc_partials, tc_partials.reshape(_NTC, _S),
      w[:_NSC], w[_NSC:],
      threshold.reshape(1, 1), temp.reshape(1, 1))

    return (out.reshape(1, _S, _D), attention_mask)


# speedup vs baseline: 1.8709x; 1.1060x over previous
import jax
import jax.numpy as jnp
from jax.experimental import pallas as pl
from jax.experimental.pallas import tpu as pltpu

_H = 12
_S = 2048
_D = 768
_BLK = 256
_RPH = _S // _BLK
_FB = 512


def _colsum_body(p_ref, out_ref):
    r = pl.program_id(0)

    @pl.when(r % _RPH == 0)
    def _():
        out_ref[...] = jnp.zeros_like(out_ref)

    out_ref[...] += jnp.sum(p_ref[...], axis=1, keepdims=True)


def _finalize_body(h_ref, cs_ref, hm_ref, thr_ref, temp_ref, out_ref):
    b = pl.program_id(0)
    hm = hm_ref[...]
    imp = jax.lax.dot_general(
        cs_ref[...], hm,
        dimension_numbers=(((0,), (0,)), ((), ())),
        preferred_element_type=jnp.float32,
    ) / jnp.sum(hm)
    row = jax.lax.broadcasted_iota(jnp.int32, imp.shape, 0) + b * _FB
    imp = jnp.where(row == 0, imp + 100.0, imp)
    mask = jax.nn.sigmoid((imp - thr_ref[0, 0]) / temp_ref[0, 0])
    out_ref[...] = h_ref[...] * mask


def kernel(hidden_states, attention_probs, head_masks, attention_mask, temp, threshold):
    probs = attention_probs.reshape(_H * _RPH, _BLK, _S)

    colsums = pl.pallas_call(
        _colsum_body,
        grid=(_H * _RPH,),
        in_specs=[pl.BlockSpec((1, _BLK, _S), lambda r: (r, 0, 0))],
        out_specs=pl.BlockSpec((1, 1, _S), lambda r: (r // _RPH, 0, 0)),
        out_shape=jax.ShapeDtypeStruct((_H, 1, _S), jnp.float32),
        compiler_params=pltpu.CompilerParams(
            dimension_semantics=("arbitrary",),
        ),
    )(probs)

    hidden = hidden_states.reshape(_S, _D)
    out = pl.pallas_call(
        _finalize_body,
        grid=(_S // _FB,),
        in_specs=[
            pl.BlockSpec((_FB, _D), lambda b: (b, 0)),
            pl.BlockSpec((_H, _FB), lambda b: (0, b)),
            pl.BlockSpec((_H, 1), lambda b: (0, 0)),
            pl.BlockSpec((1, 1), lambda b: (0, 0)),
            pl.BlockSpec((1, 1), lambda b: (0, 0)),
        ],
        out_specs=pl.BlockSpec((_FB, _D), lambda b: (b, 0)),
        out_shape=jax.ShapeDtypeStruct((_S, _D), jnp.float32),
        compiler_params=pltpu.CompilerParams(
            dimension_semantics=("arbitrary",),
        ),
    )(hidden, colsums.reshape(_H, _S), head_masks.reshape(_H, 1),
      threshold.reshape(1, 1), temp.reshape(1, 1))

    return (out.reshape(1, _S, _D), attention_mask)


# speedup vs baseline: 2.6084x; 1.3942x over previous
import jax
import jax.numpy as jnp
from jax.experimental import pallas as pl
from jax.experimental.pallas import tpu as pltpu

_H = 12
_S = 2048
_D = 768
_BLK = 2048
_RPH = _S // _BLK
_FB = 512


def _colsum_body(p_ref, out_ref):
    out_ref[...] = jnp.sum(p_ref[...], axis=1, keepdims=True)


def _finalize_body(h_ref, cs_ref, hm_ref, thr_ref, temp_ref, out_ref):
    b = pl.program_id(0)
    hm = hm_ref[...]
    imp = jax.lax.dot_general(
        cs_ref[...], hm,
        dimension_numbers=(((0,), (0,)), ((), ())),
        preferred_element_type=jnp.float32,
    ) / jnp.sum(hm)
    row = jax.lax.broadcasted_iota(jnp.int32, imp.shape, 0) + b * _FB
    imp = jnp.where(row == 0, imp + 100.0, imp)
    mask = jax.nn.sigmoid((imp - thr_ref[0, 0]) / temp_ref[0, 0])
    out_ref[...] = h_ref[...] * mask


def kernel(hidden_states, attention_probs, head_masks, attention_mask, temp, threshold):
    probs = attention_probs.reshape(_H * _RPH, _BLK, _S)

    colsums = pl.pallas_call(
        _colsum_body,
        grid=(_H * _RPH,),
        in_specs=[pl.BlockSpec((1, _BLK, _S), lambda r: (r, 0, 0))],
        out_specs=pl.BlockSpec((1, 1, _S), lambda r: (r, 0, 0)),
        out_shape=jax.ShapeDtypeStruct((_H, 1, _S), jnp.float32),
        compiler_params=pltpu.CompilerParams(
            dimension_semantics=("parallel",),
        ),
    )(probs)

    hidden = hidden_states.reshape(_S, _D)
    out = pl.pallas_call(
        _finalize_body,
        grid=(_S // _FB,),
        in_specs=[
            pl.BlockSpec((_FB, _D), lambda b: (b, 0)),
            pl.BlockSpec((_H, _FB), lambda b: (0, b)),
            pl.BlockSpec((_H, 1), lambda b: (0, 0)),
            pl.BlockSpec((1, 1), lambda b: (0, 0)),
            pl.BlockSpec((1, 1), lambda b: (0, 0)),
        ],
        out_specs=pl.BlockSpec((_FB, _D), lambda b: (b, 0)),
        out_shape=jax.ShapeDtypeStruct((_S, _D), jnp.float32),
        compiler_params=pltpu.CompilerParams(
            dimension_semantics=("arbitrary",),
        ),
    )(hidden, colsums.reshape(_H, _S), head_masks.reshape(_H, 1),
      threshold.reshape(1, 1), temp.reshape(1, 1))

    return (out.reshape(1, _S, _D), attention_mask)


# speedup vs baseline: 2.6939x; 1.0328x over previous
import jax
import jax.numpy as jnp
from jax.experimental import pallas as pl
from jax.experimental.pallas import tpu as pltpu

_H = 12
_S = 2048
_D = 768
_FB = 512
_NF = _S // _FB


def _fused_body(p_ref, h_ref, hm_ref, thr_ref, temp_ref, out_ref, acc):
    r = pl.program_id(0)

    @pl.when(r < _H)
    def _():
        acc[pl.ds(r, 1), :] = jnp.sum(p_ref[0], axis=0, keepdims=True)

    @pl.when(r >= _H)
    def _():
        b = r - _H
        hm = hm_ref[...]
        cs = acc[:, pl.ds(b * _FB, _FB)]
        imp = jax.lax.dot_general(
            cs, hm,
            dimension_numbers=(((0,), (0,)), ((), ())),
            preferred_element_type=jnp.float32,
        ) / jnp.sum(hm)
        row = jax.lax.broadcasted_iota(jnp.int32, imp.shape, 0) + b * _FB
        imp = jnp.where(row == 0, imp + 100.0, imp)
        mask = jax.nn.sigmoid((imp - thr_ref[0, 0]) / temp_ref[0, 0])
        out_ref[...] = h_ref[...] * mask


def kernel(hidden_states, attention_probs, head_masks, attention_mask, temp, threshold):
    probs = attention_probs.reshape(_H, _S, _S)
    hidden = hidden_states.reshape(_S, _D)

    out = pl.pallas_call(
        _fused_body,
        grid=(_H + _NF,),
        in_specs=[
            pl.BlockSpec((1, _S, _S),
                         lambda r: (jnp.minimum(r, _H - 1), 0, 0)),
            pl.BlockSpec((_FB, _D),
                         lambda r: (jnp.maximum(r - _H, 0), 0)),
            pl.BlockSpec((_H, 1), lambda r: (0, 0)),
            pl.BlockSpec((1, 1), lambda r: (0, 0)),
            pl.BlockSpec((1, 1), lambda r: (0, 0)),
        ],
        out_specs=pl.BlockSpec((_FB, _D), lambda r: (jnp.maximum(r - _H, 0), 0)),
        out_shape=jax.ShapeDtypeStruct((_S, _D), jnp.float32),
        scratch_shapes=[pltpu.VMEM((_H, _S), jnp.float32)],
        compiler_params=pltpu.CompilerParams(
            dimension_semantics=("arbitrary",),
        ),
    )(probs, hidden, head_masks.reshape(_H, 1),
      threshold.reshape(1, 1), temp.reshape(1, 1))

    return (out.reshape(1, _S, _D), attention_mask)
